# baseline (speedup 1.0000x reference)
import jax
import jax.numpy as jnp
from jax.experimental import pallas as pl

_N_LEVELS = 4
_BETA = 0.25
_TBLK = 256
_KC = 512


def _rvq_body(x_ref, cb_ref, cbsq_ref, xq_ref, idx_ref, loss_ref):
    t = pl.program_id(0)

    @pl.when(t == 0)
    def _init():
        loss_ref[...] = jnp.zeros_like(loss_ref)

    residual = x_ref[...]
    T, D = residual.shape
    K = cb_ref.shape[1]
    n_chunks = K // _KC
    xq = jnp.zeros_like(residual)
    sq_cols = jnp.zeros((1, D), dtype=jnp.float32)
    ident = (jax.lax.broadcasted_iota(jnp.int32, (T, T), 0)
             == jax.lax.broadcasted_iota(jnp.int32, (T, T), 1)
             ).astype(jnp.float32)
    for i in range(_N_LEVELS):
        rsq = jnp.sum(residual ** 2, axis=1, keepdims=True)
        best = jnp.full((T, 1), jnp.inf, dtype=jnp.float32)
        bidx = jnp.zeros((T, 1), dtype=jnp.int32)
        for c in range(n_chunks):
            cb_c = cb_ref[i, c * _KC:(c + 1) * _KC, :]
            cbsq = cbsq_ref[i, :, c * _KC:(c + 1) * _KC]
            dot = jax.lax.dot_general(
                residual, cb_c, (((1,), (1,)), ((), ())),
                preferred_element_type=jnp.float32)
            d = rsq + cbsq - 2.0 * dot
            m = jnp.min(d, axis=1, keepdims=True)
            a = jnp.argmin(d, axis=1, keepdims=True).astype(jnp.int32)
            upd = m < best
            bidx = jnp.where(upd, a + c * _KC, bidx)
            best = jnp.where(upd, m, best)
        zq = jnp.zeros_like(residual)
        for c in range(n_chunks):
            onehot = (jax.lax.broadcasted_iota(jnp.int32, (T, _KC), 1)
                      + c * _KC == bidx).astype(jnp.float32)
            zq = zq + jax.lax.dot_general(
                onehot, cb_ref[i, c * _KC:(c + 1) * _KC, :],
                (((1,), (0,)), ((), ())),
                precision=jax.lax.Precision.HIGHEST,
                preferred_element_type=jnp.float32)
        err = zq - residual
        sq_cols = sq_cols + jnp.sum(err * err, axis=0, keepdims=True)
        bidx_lane = jax.lax.dot_general(
            bidx.astype(jnp.float32), ident, (((0,), (0,)), ((), ())),
            precision=jax.lax.Precision.HIGHEST,
            preferred_element_type=jnp.float32)
        idx_ref[i, :, :] = bidx_lane.astype(jnp.int32)
        xres = residual + err
        residual = residual - xres
        xq = xq + xres
    xq_ref[...] = xq
    row = jax.lax.broadcasted_iota(jnp.int32, loss_ref.shape, 0)
    loss_ref[...] += jnp.where(
        row == 0, jnp.broadcast_to(sq_cols, loss_ref.shape), 0.0)


def kernel(x, codebooks):
    B, T, D = x.shape
    L, K, _ = codebooks.shape
    N = B * T
    flat = x.reshape(N, D)
    grid = (N // _TBLK,)
    cbsq = jnp.stack([jnp.sum(codebooks[i] ** 2, axis=1)
                      for i in range(L)])[:, None, :]

    xq, idx, loss = pl.pallas_call(
        _rvq_body,
        grid=grid,
        in_specs=[
            pl.BlockSpec((_TBLK, D), lambda t: (t, 0)),
            pl.BlockSpec((L, K, D), lambda t: (0, 0, 0)),
            pl.BlockSpec((L, 1, K), lambda t: (0, 0, 0)),
        ],
        out_specs=[
            pl.BlockSpec((_TBLK, D), lambda t: (t, 0)),
            pl.BlockSpec((L, 1, _TBLK), lambda t: (0, 0, t)),
            pl.BlockSpec((8, D), lambda t: (0, 0)),
        ],
        out_shape=[
            jax.ShapeDtypeStruct((N, D), jnp.float32),
            jax.ShapeDtypeStruct((L, 1, N), jnp.int32),
            jax.ShapeDtypeStruct((8, D), jnp.float32),
        ],
    )(flat, codebooks, cbsq)

    x_q = xq.reshape(B, T, D)
    indices = jnp.moveaxis(idx.reshape(L, B, T), 0, -1)
    mean_losses = jnp.sum(loss) * (1.0 + _BETA) / (L * N * D)
    return (x_q, mean_losses, indices)


# speedup vs baseline: 1.4787x; 1.4787x over previous
import jax
import jax.numpy as jnp
from jax.experimental import pallas as pl

_N_LEVELS = 4
_BETA = 0.25
_TBLK = 256
_KC = 512


def _rvq_body(x_ref, cb_ref, cbsq_ref, cbp_ref, xq_ref, idx_ref, loss_ref):
    t = pl.program_id(0)

    @pl.when(t == 0)
    def _init():
        loss_ref[...] = jnp.zeros_like(loss_ref)

    residual = x_ref[...]
    T, D = residual.shape
    K = cb_ref.shape[1]
    n_chunks = K // _KC
    xq = jnp.zeros_like(residual)
    sq_cols = jnp.zeros((1, D), dtype=jnp.float32)
    ident = (jax.lax.broadcasted_iota(jnp.int32, (T, T), 0)
             == jax.lax.broadcasted_iota(jnp.int32, (T, T), 1)
             ).astype(jnp.float32)
    for i in range(_N_LEVELS):
        rsq = jnp.sum(residual ** 2, axis=1, keepdims=True)
        best = jnp.full((T, 1), jnp.inf, dtype=jnp.float32)
        bidx = jnp.zeros((T, 1), dtype=jnp.int32)
        for c in range(n_chunks):
            cb_c = cb_ref[i, c * _KC:(c + 1) * _KC, :]
            cbsq = cbsq_ref[i, :, c * _KC:(c + 1) * _KC]
            dot = jax.lax.dot_general(
                residual, cb_c, (((1,), (1,)), ((), ())),
                preferred_element_type=jnp.float32)
            d = rsq + cbsq - 2.0 * dot
            m = jnp.min(d, axis=1, keepdims=True)
            a = jnp.argmin(d, axis=1, keepdims=True).astype(jnp.int32)
            upd = m < best
            bidx = jnp.where(upd, a + c * _KC, bidx)
            best = jnp.where(upd, m, best)
        zq = jnp.zeros_like(residual)
        for c in range(n_chunks):
            onehot = (jax.lax.broadcasted_iota(jnp.int32, (T, _KC), 1)
                      + c * _KC == bidx).astype(jnp.bfloat16)
            zq_c = jnp.zeros_like(residual)
            for p in range(3):
                zq_c = zq_c + jax.lax.dot_general(
                    onehot, cbp_ref[p, i, c * _KC:(c + 1) * _KC, :],
                    (((1,), (0,)), ((), ())),
                    preferred_element_type=jnp.float32)
            zq = zq + zq_c
        err = zq - residual
        sq_cols = sq_cols + jnp.sum(err * err, axis=0, keepdims=True)
        bidx_lane = jax.lax.dot_general(
            bidx.astype(jnp.float32), ident, (((0,), (0,)), ((), ())),
            precision=jax.lax.Precision.HIGHEST,
            preferred_element_type=jnp.float32)
        idx_ref[i, :, :] = bidx_lane.astype(jnp.int32)
        xres = residual + err
        residual = residual - xres
        xq = xq + xres
    xq_ref[...] = xq
    row = jax.lax.broadcasted_iota(jnp.int32, loss_ref.shape, 0)
    loss_ref[...] += jnp.where(
        row == 0, jnp.broadcast_to(sq_cols, loss_ref.shape), 0.0)


def kernel(x, codebooks):
    B, T, D = x.shape
    L, K, _ = codebooks.shape
    N = B * T
    flat = x.reshape(N, D)
    grid = (N // _TBLK,)
    cbsq = jnp.stack([jnp.sum(codebooks[i] ** 2, axis=1)
                      for i in range(L)])[:, None, :]
    mask = jnp.uint32(0xFFFF0000)
    p1f = jax.lax.bitcast_convert_type(
        jax.lax.bitcast_convert_type(codebooks, jnp.uint32) & mask,
        jnp.float32)
    r1 = codebooks - p1f
    p2f = jax.lax.bitcast_convert_type(
        jax.lax.bitcast_convert_type(r1, jnp.uint32) & mask, jnp.float32)
    r2 = r1 - p2f
    cbp = jnp.stack([p1f.astype(jnp.bfloat16),
                     p2f.astype(jnp.bfloat16),
                     r2.astype(jnp.bfloat16)])

    xq, idx, loss = pl.pallas_call(
        _rvq_body,
        grid=grid,
        in_specs=[
            pl.BlockSpec((_TBLK, D), lambda t: (t, 0)),
            pl.BlockSpec((L, K, D), lambda t: (0, 0, 0)),
            pl.BlockSpec((L, 1, K), lambda t: (0, 0, 0)),
            pl.BlockSpec((3, L, K, D), lambda t: (0, 0, 0, 0)),
        ],
        out_specs=[
            pl.BlockSpec((_TBLK, D), lambda t: (t, 0)),
            pl.BlockSpec((L, 1, _TBLK), lambda t: (0, 0, t)),
            pl.BlockSpec((8, D), lambda t: (0, 0)),
        ],
        out_shape=[
            jax.ShapeDtypeStruct((N, D), jnp.float32),
            jax.ShapeDtypeStruct((L, 1, N), jnp.int32),
            jax.ShapeDtypeStruct((8, D), jnp.float32),
        ],
    )(flat, codebooks, cbsq, cbp)

    x_q = xq.reshape(B, T, D)
    indices = jnp.moveaxis(idx.reshape(L, B, T), 0, -1)
    mean_losses = jnp.sum(loss) * (1.0 + _BETA) / (L * N * D)
    return (x_q, mean_losses, indices)
